# Initial kernel scaffold; baseline (speedup 1.0000x reference)
import jax
import jax.numpy as jnp
from jax.experimental import pallas as pl


def kernel(x, emb_weight):
    raise NotImplementedError("write your pallas kernel here")


# speedup vs baseline: 1.0015x; 1.0015x over previous
import jax
import jax.numpy as jnp
from jax.experimental import pallas as pl


_SEQ_BLOCK = 512


def _add_kernel(x_ref, emb_ref, out_ref):
    out_ref[...] = x_ref[...] + emb_ref[...][None, :, :]


def kernel(x, emb_weight):
    bs, seq_len, ch = x.shape
    emb = emb_weight[:seq_len]
    blk = _SEQ_BLOCK if seq_len % _SEQ_BLOCK == 0 else seq_len
    grid = (seq_len // blk,)
    return pl.pallas_call(
        _add_kernel,
        grid=grid,
        in_specs=[
            pl.BlockSpec((bs, blk, ch), lambda i: (0, i, 0)),
            pl.BlockSpec((blk, ch), lambda i: (i, 0)),
        ],
        out_specs=pl.BlockSpec((bs, blk, ch), lambda i: (0, i, 0)),
        out_shape=jax.ShapeDtypeStruct((bs, seq_len, ch), x.dtype),
    )(x, emb)
